# Initial kernel scaffold; baseline (speedup 1.0000x reference)
import jax
import jax.numpy as jnp
from jax.experimental import pallas as pl


def kernel(x, edge_index, W1l, b1l, W1r, W2l, b2l, W2r, W3l, b3l, W3r, Wc, bc):
    raise NotImplementedError("write your pallas kernel here")


# speedup vs baseline: 2.9444x; 2.9444x over previous
import jax
import jax.numpy as jnp
from jax import lax
from jax.experimental import pallas as pl
from jax.experimental.pallas import tpu as pltpu
from jax.experimental.pallas import tpu_sc as plsc

N = 10000
E = 320000
D = 128
D_EXT = 144

NC = 2
NS = 16
NW = NC * NS
CHUNK = 64
NCH = 160
E_PAD = NW * NCH * CHUNK
KI = 8
NSUP = NCH // KI
NPAD = 10112
ROWS_PER = NPAD // NS
QFULL = ROWS_PER // CHUNK
QREM = ROWS_PER - QFULL * CHUNK

_MESH = plsc.VectorSubcoreMesh(core_axis_name="c", subcore_axis_name="s")


def _make_agg(width):
    def body(h, srcI, dstI, zrows, agg_out, src_v, dst_v, rows_v, acc, sem):
        cid = lax.axis_index("c")
        sid = lax.axis_index("s")
        wid = cid * NS + sid
        base = sid * ROWS_PER

        pltpu.sync_copy(zrows, rows_v)
        for q in range(QFULL):
            pltpu.sync_copy(rows_v, acc.at[pl.ds(base + q * CHUNK, CHUNK)])
        pltpu.sync_copy(rows_v.at[pl.ds(0, QREM)],
                        acc.at[pl.ds(base + QFULL * CHUNK, QREM)])
        plsc.subcore_barrier()

        @pl.loop(0, NSUP)
        def _(t):
            pltpu.sync_copy(srcI.at[wid, pl.ds(t * KI, KI)], src_v)
            pltpu.sync_copy(dstI.at[wid, pl.ds(t * KI, KI)], dst_v)
            for j in range(KI):
                pltpu.async_copy(h.at[src_v.at[j]], rows_v, sem).wait()
                pltpu.sync_copy(rows_v, acc.at[dst_v.at[j]], add=True)

        plsc.subcore_barrier()
        for q in range(QFULL):
            sl = pl.ds(base + q * CHUNK, CHUNK)
            pltpu.sync_copy(acc.at[sl], rows_v)
            pltpu.sync_copy(rows_v, agg_out.at[cid, sl])
        slr = pl.ds(base + QFULL * CHUNK, QREM)
        pltpu.sync_copy(acc.at[slr], rows_v.at[pl.ds(0, QREM)])
        pltpu.sync_copy(rows_v.at[pl.ds(0, QREM)], agg_out.at[cid, slr])

    return pl.kernel(
        body,
        out_type=jax.ShapeDtypeStruct((NC, NPAD, width), jnp.float32),
        mesh=_MESH,
        scratch_types=[
            pltpu.VMEM((KI, CHUNK), jnp.int32),
            pltpu.VMEM((KI, CHUNK), jnp.int32),
            pltpu.VMEM((CHUNK, width), jnp.float32),
            pltpu.VMEM_SHARED((NPAD, width), jnp.float32),
            pltpu.SemaphoreType.DMA,
        ],
    )


_agg_call = _make_agg(D)


def _cnt_body(dstI, zrows, orows, cnt_out, dst_v, buf_v, acc):
    cid = lax.axis_index("c")
    sid = lax.axis_index("s")
    wid = cid * NS + sid
    base = sid * ROWS_PER

    pltpu.sync_copy(zrows, buf_v)
    for q in range(QFULL):
        pltpu.sync_copy(buf_v, acc.at[pl.ds(base + q * CHUNK, CHUNK)])
    pltpu.sync_copy(buf_v.at[pl.ds(0, QREM)],
                    acc.at[pl.ds(base + QFULL * CHUNK, QREM)])
    pltpu.sync_copy(orows, buf_v)
    plsc.subcore_barrier()

    @pl.loop(0, NSUP)
    def _(t):
        pltpu.sync_copy(dstI.at[wid, pl.ds(t * KI, KI)], dst_v)
        for j in range(KI):
            pltpu.sync_copy(buf_v, acc.at[dst_v.at[j]], add=True)

    plsc.subcore_barrier()
    for q in range(QFULL):
        sl = pl.ds(base + q * CHUNK, CHUNK)
        pltpu.sync_copy(acc.at[sl], buf_v)
        pltpu.sync_copy(buf_v, cnt_out.at[cid, sl])
    slr = pl.ds(base + QFULL * CHUNK, QREM)
    pltpu.sync_copy(acc.at[slr], buf_v.at[pl.ds(0, QREM)])
    pltpu.sync_copy(buf_v.at[pl.ds(0, QREM)], cnt_out.at[cid, slr])


_cnt_call = pl.kernel(
    _cnt_body,
    out_type=jax.ShapeDtypeStruct((NC, NPAD, D), jnp.float32),
    mesh=_MESH,
    scratch_types=[
        pltpu.VMEM((KI, CHUNK), jnp.int32),
        pltpu.VMEM((CHUNK, D), jnp.float32),
        pltpu.VMEM_SHARED((NPAD, D), jnp.float32),
    ],
)


BLK = 1000
GRID = N // BLK


def _layer_body(p0, p1, c0, c1, h, wl, bl, wr, out):
    inv = 1.0 / jnp.maximum(c0[...] + c1[...], 1.0)
    agg = (p0[...] + p1[...]) * inv
    y = (jnp.dot(agg, wl[...], preferred_element_type=jnp.float32)
         + bl[...]
         + jnp.dot(h[...], wr[...], preferred_element_type=jnp.float32))
    out[...] = jnp.maximum(y, 0.0)


def _final_body(p0, p1, c0, c1, h, wl, bl, wr, wc, bc, out):
    inv = 1.0 / jnp.maximum(c0[...] + c1[...], 1.0)
    agg = (p0[...] + p1[...]) * inv
    y = (jnp.dot(agg, wl[...], preferred_element_type=jnp.float32)
         + bl[...]
         + jnp.dot(h[...], wr[...], preferred_element_type=jnp.float32))
    y = jnp.maximum(y, 0.0)
    z = jnp.dot(y, wc[...], preferred_element_type=jnp.float32) + bc[...]
    out[...] = jax.nn.sigmoid(z)


def _row_spec(w):
    return pl.BlockSpec((BLK, w), lambda i: (i, 0))


def _full_spec(r, c):
    return pl.BlockSpec((r, c), lambda i: (0, 0))


_layer_call = pl.pallas_call(
    _layer_body,
    grid=(GRID,),
    in_specs=[
        _row_spec(D), _row_spec(D), _row_spec(1), _row_spec(1), _row_spec(D),
        _full_spec(D, D), _full_spec(1, D), _full_spec(D, D),
    ],
    out_specs=_row_spec(D),
    out_shape=jax.ShapeDtypeStruct((N, D), jnp.float32),
)

_final_call = pl.pallas_call(
    _final_body,
    grid=(GRID,),
    in_specs=[
        _row_spec(D), _row_spec(D), _row_spec(1), _row_spec(1), _row_spec(D),
        _full_spec(D, D), _full_spec(1, D), _full_spec(D, D),
        _full_spec(D, 1), _full_spec(1, 1),
    ],
    out_specs=_row_spec(1),
    out_shape=jax.ShapeDtypeStruct((N, 1), jnp.float32),
)


@jax.jit
def _run(x, edge_index, W1l, b1l, W1r, W2l, b2l, W2r, W3l, b3l, W3r, Wc, bc):
    src = edge_index[0].astype(jnp.int32)
    dst = edge_index[1].astype(jnp.int32)
    pad = E_PAD - E
    src_p = jnp.concatenate([src, jnp.zeros((pad,), jnp.int32)])
    dst_p = jnp.concatenate([dst, jnp.full((pad,), N, jnp.int32)])
    srcI = src_p.reshape(NW, NCH, CHUNK)
    dstI = dst_p.reshape(NW, NCH, CHUNK)

    z_d = jnp.zeros((CHUNK, D), jnp.float32)
    o_d = jnp.ones((CHUNK, D), jnp.float32)

    cnt2 = _cnt_call(dstI, z_d, o_d)
    c0 = cnt2[0, :N, :1]
    c1 = cnt2[1, :N, :1]

    agg1 = _agg_call(x, srcI, dstI, z_d)
    h1 = _layer_call(agg1[0, :N], agg1[1, :N], c0, c1, x,
                     W1l.T, b1l.reshape(1, D), W1r.T)
    agg2 = _agg_call(h1, srcI, dstI, z_d)
    h2 = _layer_call(agg2[0, :N], agg2[1, :N], c0, c1, h1,
                     W2l.T, b2l.reshape(1, D), W2r.T)
    agg3 = _agg_call(h2, srcI, dstI, z_d)
    out = _final_call(agg3[0, :N], agg3[1, :N], c0, c1, h2,
                      W3l.T, b3l.reshape(1, D), W3r.T,
                      Wc.T, bc.reshape(1, 1))
    return out[:, 0]


def kernel(x, edge_index, W1l, b1l, W1r, W2l, b2l, W2r, W3l, b3l, W3r, Wc, bc):
    return _run(x, edge_index, W1l, b1l, W1r, W2l, b2l, W2r, W3l, b3l, W3r,
                Wc, bc)


# speedup vs baseline: 3.2369x; 1.0993x over previous
import jax
import jax.numpy as jnp
from jax import lax
from jax.experimental import pallas as pl
from jax.experimental.pallas import tpu as pltpu
from jax.experimental.pallas import tpu_sc as plsc

N = 10000
E = 320000
D = 128
D_EXT = 144

NC = 2
NS = 16
NW = NC * NS
CHUNK = 80
NCH = 128
E_PAD = NW * NCH * CHUNK
KI = 4
NSUP = NCH // KI
HALF = NSUP // 2
NPAD = 10112
ROWS_PER = NPAD // NS
QFULL = ROWS_PER // CHUNK
QREM = ROWS_PER - QFULL * CHUNK

_MESH = plsc.VectorSubcoreMesh(core_axis_name="c", subcore_axis_name="s")


def _make_agg(width):
    def body(h, srcI, dstI, zrows, agg_out,
             src_v, dst_v, rows_v, acc, sg0, sg1, si0, si1):
        cid = lax.axis_index("c")
        sid = lax.axis_index("s")
        wid = cid * NS + sid
        base = sid * ROWS_PER
        sgs = (sg0, sg1)
        sis = (si0, si1)

        def stage_idx(s, slot, sem):
            pltpu.async_copy(srcI.at[wid, pl.ds(s * KI, KI)],
                             src_v.at[slot], sem)
            pltpu.async_copy(dstI.at[wid, pl.ds(s * KI, KI)],
                             dst_v.at[slot], sem)

        def wait_idx(slot, sem):
            pltpu.make_async_copy(srcI.at[wid, pl.ds(0, KI)],
                                  src_v.at[slot], sem).wait()
            pltpu.make_async_copy(dstI.at[wid, pl.ds(0, KI)],
                                  dst_v.at[slot], sem).wait()

        def start_gather(slot, j, buf):
            pltpu.async_copy(h.at[src_v.at[slot, j]], rows_v.at[buf],
                             sgs[buf])

        def wait_gather(buf):
            pltpu.make_async_copy(h.at[pl.ds(0, CHUNK)], rows_v.at[buf],
                                  sgs[buf]).wait()

        pltpu.sync_copy(zrows, rows_v.at[0])
        for q in range(QFULL):
            pltpu.sync_copy(rows_v.at[0],
                            acc.at[pl.ds(base + q * CHUNK, CHUNK)])
        pltpu.sync_copy(rows_v.at[0, pl.ds(0, QREM)],
                        acc.at[pl.ds(base + QFULL * CHUNK, QREM)])
        plsc.subcore_barrier()

        stage_idx(0, 0, si0)
        wait_idx(0, si0)
        start_gather(0, 0, 0)

        def run_super(slot, nxt_stage, nxt_sem, have_next):
            nxt_stage()
            for j in range(KI):
                cur = j % 2
                if j + 1 < KI:
                    start_gather(slot, j + 1, (j + 1) % 2)
                elif have_next is True:
                    wait_idx(1 - slot, nxt_sem)
                    start_gather(1 - slot, 0, (j + 1) % 2)
                else:
                    @pl.when(have_next)
                    def _():
                        wait_idx(1 - slot, nxt_sem)
                        start_gather(1 - slot, 0, (j + 1) % 2)
                wait_gather(cur)
                pltpu.sync_copy(rows_v.at[cur], acc.at[dst_v.at[slot, j]],
                                add=True)

        @pl.loop(0, HALF)
        def _(u):
            s0 = 2 * u
            run_super(0, lambda: stage_idx(s0 + 1, 1, si1), si1, True)
            pred = s0 + 2 < NSUP

            def stage_next():
                @pl.when(pred)
                def _():
                    stage_idx(s0 + 2, 0, si0)

            run_super(1, stage_next, si0, pred)

        plsc.subcore_barrier()
        for q in range(QFULL):
            sl = pl.ds(base + q * CHUNK, CHUNK)
            pltpu.sync_copy(acc.at[sl], rows_v.at[0])
            pltpu.sync_copy(rows_v.at[0], agg_out.at[cid, sl])
        slr = pl.ds(base + QFULL * CHUNK, QREM)
        pltpu.sync_copy(acc.at[slr], rows_v.at[0, pl.ds(0, QREM)])
        pltpu.sync_copy(rows_v.at[0, pl.ds(0, QREM)], agg_out.at[cid, slr])

    return pl.kernel(
        body,
        out_type=jax.ShapeDtypeStruct((NC, NPAD, width), jnp.float32),
        mesh=_MESH,
        scratch_types=[
            pltpu.VMEM((2, KI, CHUNK), jnp.int32),
            pltpu.VMEM((2, KI, CHUNK), jnp.int32),
            pltpu.VMEM((2, CHUNK, width), jnp.float32),
            pltpu.VMEM_SHARED((NPAD, width), jnp.float32),
            pltpu.SemaphoreType.DMA,
            pltpu.SemaphoreType.DMA,
            pltpu.SemaphoreType.DMA,
            pltpu.SemaphoreType.DMA,
        ],
    )


_agg_call = _make_agg(D)


def _cnt_body(dstI, zrows, orows, cnt_out, dst_v, buf_v, acc):
    cid = lax.axis_index("c")
    sid = lax.axis_index("s")
    wid = cid * NS + sid
    base = sid * ROWS_PER

    pltpu.sync_copy(zrows, buf_v)
    for q in range(QFULL):
        pltpu.sync_copy(buf_v, acc.at[pl.ds(base + q * CHUNK, CHUNK)])
    pltpu.sync_copy(buf_v.at[pl.ds(0, QREM)],
                    acc.at[pl.ds(base + QFULL * CHUNK, QREM)])
    pltpu.sync_copy(orows, buf_v)
    plsc.subcore_barrier()

    @pl.loop(0, NSUP)
    def _(t):
        pltpu.sync_copy(dstI.at[wid, pl.ds(t * KI, KI)], dst_v)
        for j in range(KI):
            pltpu.sync_copy(buf_v, acc.at[dst_v.at[j]], add=True)

    plsc.subcore_barrier()
    for q in range(QFULL):
        sl = pl.ds(base + q * CHUNK, CHUNK)
        pltpu.sync_copy(acc.at[sl], buf_v)
        pltpu.sync_copy(buf_v, cnt_out.at[cid, sl])
    slr = pl.ds(base + QFULL * CHUNK, QREM)
    pltpu.sync_copy(acc.at[slr], buf_v.at[pl.ds(0, QREM)])
    pltpu.sync_copy(buf_v.at[pl.ds(0, QREM)], cnt_out.at[cid, slr])


_cnt_call = pl.kernel(
    _cnt_body,
    out_type=jax.ShapeDtypeStruct((NC, NPAD, D), jnp.float32),
    mesh=_MESH,
    scratch_types=[
        pltpu.VMEM((KI, CHUNK), jnp.int32),
        pltpu.VMEM((CHUNK, D), jnp.float32),
        pltpu.VMEM_SHARED((NPAD, D), jnp.float32),
    ],
)


BLK = 1000
GRID = N // BLK


def _layer_body(p0, p1, c0, c1, h, wl, bl, wr, out):
    inv = 1.0 / jnp.maximum(c0[...] + c1[...], 1.0)
    agg = (p0[...] + p1[...]) * inv
    y = (jnp.dot(agg, wl[...], preferred_element_type=jnp.float32)
         + bl[...]
         + jnp.dot(h[...], wr[...], preferred_element_type=jnp.float32))
    out[...] = jnp.maximum(y, 0.0)


def _final_body(p0, p1, c0, c1, h, wl, bl, wr, wc, bc, out):
    inv = 1.0 / jnp.maximum(c0[...] + c1[...], 1.0)
    agg = (p0[...] + p1[...]) * inv
    y = (jnp.dot(agg, wl[...], preferred_element_type=jnp.float32)
         + bl[...]
         + jnp.dot(h[...], wr[...], preferred_element_type=jnp.float32))
    y = jnp.maximum(y, 0.0)
    z = jnp.dot(y, wc[...], preferred_element_type=jnp.float32) + bc[...]
    out[...] = jax.nn.sigmoid(z)


def _row_spec(w):
    return pl.BlockSpec((BLK, w), lambda i: (i, 0))


def _full_spec(r, c):
    return pl.BlockSpec((r, c), lambda i: (0, 0))


_layer_call = pl.pallas_call(
    _layer_body,
    grid=(GRID,),
    in_specs=[
        _row_spec(D), _row_spec(D), _row_spec(1), _row_spec(1), _row_spec(D),
        _full_spec(D, D), _full_spec(1, D), _full_spec(D, D),
    ],
    out_specs=_row_spec(D),
    out_shape=jax.ShapeDtypeStruct((N, D), jnp.float32),
)

_final_call = pl.pallas_call(
    _final_body,
    grid=(GRID,),
    in_specs=[
        _row_spec(D), _row_spec(D), _row_spec(1), _row_spec(1), _row_spec(D),
        _full_spec(D, D), _full_spec(1, D), _full_spec(D, D),
        _full_spec(D, 1), _full_spec(1, 1),
    ],
    out_specs=_row_spec(1),
    out_shape=jax.ShapeDtypeStruct((N, 1), jnp.float32),
)


@jax.jit
def _run(x, edge_index, W1l, b1l, W1r, W2l, b2l, W2r, W3l, b3l, W3r, Wc, bc):
    src = edge_index[0].astype(jnp.int32)
    dst = edge_index[1].astype(jnp.int32)
    pad = E_PAD - E
    src_p = jnp.concatenate([src, jnp.zeros((pad,), jnp.int32)])
    dst_p = jnp.concatenate([dst, jnp.full((pad,), N, jnp.int32)])
    srcI = src_p.reshape(NW, NCH, CHUNK)
    dstI = dst_p.reshape(NW, NCH, CHUNK)

    z_d = jnp.zeros((CHUNK, D), jnp.float32)
    o_d = jnp.ones((CHUNK, D), jnp.float32)

    cnt2 = _cnt_call(dstI, z_d, o_d)
    c0 = cnt2[0, :N, :1]
    c1 = cnt2[1, :N, :1]

    agg1 = _agg_call(x, srcI, dstI, z_d)
    h1 = _layer_call(agg1[0, :N], agg1[1, :N], c0, c1, x,
                     W1l.T, b1l.reshape(1, D), W1r.T)
    agg2 = _agg_call(h1, srcI, dstI, z_d)
    h2 = _layer_call(agg2[0, :N], agg2[1, :N], c0, c1, h1,
                     W2l.T, b2l.reshape(1, D), W2r.T)
    agg3 = _agg_call(h2, srcI, dstI, z_d)
    out = _final_call(agg3[0, :N], agg3[1, :N], c0, c1, h2,
                      W3l.T, b3l.reshape(1, D), W3r.T,
                      Wc.T, bc.reshape(1, 1))
    return out[:, 0]


def kernel(x, edge_index, W1l, b1l, W1r, W2l, b2l, W2r, W3l, b3l, W3r, Wc, bc):
    return _run(x, edge_index, W1l, b1l, W1r, W2l, b2l, W2r, W3l, b3l, W3r,
                Wc, bc)
